# baseline (speedup 1.0000x reference)
import functools

import jax
import jax.numpy as jnp
from jax import lax
from jax.experimental import pallas as pl
from jax.experimental.pallas import tpu as pltpu
from jax.experimental.pallas import tpu_sc as plsc

N_NODES = 10000
N_EDGES = 320000
N_RELS = 100
D = 128
LANES = 16

NC = 2
NS = 16
NW = NC * NS
CH = 80
N_CHUNKS = 125
SLAB_CHUNKS = 128
SUP = 16
N_SUP = 8
E_PER_TILE = N_CHUNKS * CH
ROWS_PER_TILE = 624
LAST_TILE_ROWS = N_NODES - (NS - 1) * ROWS_PER_TILE


def _sc_agg_body(src_hbm, dst_hbm, typ_hbm, emb_hbm, rel_hbm, zagg_hbm, zdeg_hbm,
                 agg_out, deg_out,
                 idx_src, idx_dst, idx_typ, emb_v, rel_v, msg_v, ones_v,
                 agg_sh, deg_sh, sem_g, sem_r, sem_s, sem_d):
    cid = lax.axis_index("c")
    sid = lax.axis_index("s")
    wid = cid * NS + sid

    row0 = sid * ROWS_PER_TILE

    @pl.when(sid < NS - 1)
    def _zero_agg():
        pltpu.sync_copy(zagg_hbm.at[pl.ds(row0, ROWS_PER_TILE)],
                        agg_sh.at[pl.ds(row0, ROWS_PER_TILE)])

    @pl.when(sid == NS - 1)
    def _zero_agg_last():
        pltpu.sync_copy(zagg_hbm.at[pl.ds((NS - 1) * ROWS_PER_TILE, LAST_TILE_ROWS)],
                        agg_sh.at[pl.ds((NS - 1) * ROWS_PER_TILE, LAST_TILE_ROWS)])

    @pl.when(sid == 0)
    def _zero_deg():
        pltpu.sync_copy(zdeg_hbm, deg_sh)

    for k in range(CH // LANES):
        ones_v[pl.ds(k * LANES, LANES)] = jnp.full((LANES,), 1.0, jnp.float32)

    plsc.subcore_barrier()

    def super_body(s, carry):
        c0 = s * SUP
        pltpu.sync_copy(src_hbm.at[wid, pl.ds(c0, SUP)], idx_src)
        pltpu.sync_copy(dst_hbm.at[wid, pl.ds(c0, SUP)], idx_dst)
        pltpu.sync_copy(typ_hbm.at[wid, pl.ds(c0, SUP)], idx_typ)
        n_inner = jnp.minimum(SUP, N_CHUNKS - c0)

        def chunk_body(i, c1):
            src_ids = idx_src.at[i]
            typ_ids = idx_typ.at[i]
            dst_ids = idx_dst.at[i]
            cg = pltpu.async_copy(emb_hbm.at[src_ids], emb_v, sem_g)
            cr = pltpu.async_copy(rel_hbm.at[typ_ids], rel_v, sem_r)
            cg.wait()
            cr.wait()

            def mul_body(r, c2):
                for k in range(D // LANES):
                    sl = pl.ds(k * LANES, LANES)
                    msg_v[r, sl] = emb_v[r, sl] * rel_v[r, sl]
                return c2
            lax.fori_loop(0, CH, mul_body, 0, unroll=8)

            pltpu.sync_copy(msg_v, agg_sh.at[dst_ids], add=True)
            pltpu.sync_copy(ones_v, deg_sh.at[dst_ids], add=True)
            return c1

        lax.fori_loop(0, n_inner, chunk_body, 0)
        return carry

    lax.fori_loop(0, N_SUP, super_body, 0)

    plsc.subcore_barrier()

    @pl.when(sid < NS - 1)
    def _write_agg():
        pltpu.sync_copy(agg_sh.at[pl.ds(row0, ROWS_PER_TILE)],
                        agg_out.at[cid, pl.ds(row0, ROWS_PER_TILE)])

    @pl.when(sid == NS - 1)
    def _write_agg_last():
        pltpu.sync_copy(agg_sh.at[pl.ds((NS - 1) * ROWS_PER_TILE, LAST_TILE_ROWS)],
                        agg_out.at[cid, pl.ds((NS - 1) * ROWS_PER_TILE, LAST_TILE_ROWS)])

    @pl.when(sid == 0)
    def _write_deg():
        pltpu.sync_copy(deg_sh, deg_out.at[cid])


_sc_agg = functools.partial(
    pl.kernel,
    out_type=[
        jax.ShapeDtypeStruct((NC, N_NODES, D), jnp.float32),
        jax.ShapeDtypeStruct((NC, N_NODES), jnp.float32),
    ],
    mesh=plsc.VectorSubcoreMesh(core_axis_name="c", subcore_axis_name="s"),
    scratch_types=[
        pltpu.VMEM((SUP, CH), jnp.int32),
        pltpu.VMEM((SUP, CH), jnp.int32),
        pltpu.VMEM((SUP, CH), jnp.int32),
        pltpu.VMEM((CH, D), jnp.float32),
        pltpu.VMEM((CH, D), jnp.float32),
        pltpu.VMEM((CH, D), jnp.float32),
        pltpu.VMEM((CH,), jnp.float32),
        pltpu.VMEM_SHARED((N_NODES, D), jnp.float32),
        pltpu.VMEM_SHARED((N_NODES,), jnp.float32),
        pltpu.SemaphoreType.DMA,
        pltpu.SemaphoreType.DMA,
        pltpu.SemaphoreType.DMA,
        pltpu.SemaphoreType.DMA,
    ],
)(_sc_agg_body)


ROW_BLK = 1000


def _tc_finish_body(aggp_ref, degp_ref, emb_ref, w_ref, wl_ref, b_ref, out_ref):
    agg = aggp_ref[0] + aggp_ref[1]
    deg = degp_ref[0, :, 0] + degp_ref[1, :, 0]
    norm = 1.0 / jnp.maximum(deg, 1.0)
    x = agg * norm[:, None]
    acc = jnp.dot(x, w_ref[...], preferred_element_type=jnp.float32)
    acc = acc + jnp.dot(emb_ref[...], wl_ref[...],
                        preferred_element_type=jnp.float32)
    out_ref[...] = jnp.tanh(acc + b_ref[...])


def _tc_finish(aggp, degp, emb, W, W_loop, b2d):
    grid = (N_NODES // ROW_BLK,)
    return pl.pallas_call(
        _tc_finish_body,
        grid=grid,
        in_specs=[
            pl.BlockSpec((NC, ROW_BLK, D), lambda i: (0, i, 0)),
            pl.BlockSpec((NC, ROW_BLK, 1), lambda i: (0, i, 0)),
            pl.BlockSpec((ROW_BLK, D), lambda i: (i, 0)),
            pl.BlockSpec((D, D), lambda i: (0, 0)),
            pl.BlockSpec((D, D), lambda i: (0, 0)),
            pl.BlockSpec((1, D), lambda i: (0, 0)),
        ],
        out_specs=pl.BlockSpec((ROW_BLK, D), lambda i: (i, 0)),
        out_shape=jax.ShapeDtypeStruct((N_NODES, D), jnp.float32),
    )(aggp, degp.reshape(NC, N_NODES, 1), emb, W, W_loop, b2d)


def _slab(x):
    x = x.reshape(NW, E_PER_TILE)
    x = jnp.pad(x, ((0, 0), (0, SLAB_CHUNKS * CH - E_PER_TILE)))
    return x.reshape(NW, SLAB_CHUNKS, CH)


def kernel(t, emb, edge_index, edge_type, W, W_loop, rel_emb, b):
    src = _slab(edge_index[0])
    dst = _slab(edge_index[1])
    typ = _slab(edge_type)
    zagg = jnp.zeros((N_NODES, D), jnp.float32)
    zdeg = jnp.zeros((N_NODES,), jnp.float32)
    aggp, degp = _sc_agg(src, dst, typ, emb, rel_emb, zagg, zdeg)
    return _tc_finish(aggp, degp, emb, W, W_loop, b.reshape(1, D))


# speedup vs baseline: 1.8723x; 1.8723x over previous
import functools

import jax
import jax.numpy as jnp
from jax import lax
from jax.experimental import pallas as pl
from jax.experimental.pallas import tpu as pltpu
from jax.experimental.pallas import tpu_sc as plsc

N_NODES = 10000
N_EDGES = 320000
N_RELS = 100
D = 128
LANES = 16

NC = 2
NS = 16
NW = NC * NS
CH = 80
N_CHUNKS = 125
SLAB_CHUNKS = 128
SUP = 16
N_SUP = 8
E_PER_TILE = N_CHUNKS * CH
ROWS_PER_TILE = 624
LAST_TILE_ROWS = N_NODES - (NS - 1) * ROWS_PER_TILE


def _sc_agg_body(src_hbm, dst_hbm, typ_hbm, emb_hbm, rel_hbm, zagg_hbm, zdeg_hbm,
                 agg_out, deg_out,
                 idx_src, idx_dst, idx_typ, emb_a, emb_b, rel_a, rel_b, ones_v,
                 agg_sh, deg_sh, rel_sp,
                 sem_ga, sem_gb, sem_ra, sem_rb, sem_s, sem_d):
    cid = lax.axis_index("c")
    sid = lax.axis_index("s")
    wid = cid * NS + sid

    row0 = sid * ROWS_PER_TILE

    @pl.when(sid < NS - 1)
    def _zero_agg():
        pltpu.sync_copy(zagg_hbm.at[pl.ds(row0, ROWS_PER_TILE)],
                        agg_sh.at[pl.ds(row0, ROWS_PER_TILE)])

    @pl.when(sid == NS - 1)
    def _zero_agg_last():
        pltpu.sync_copy(zagg_hbm.at[pl.ds((NS - 1) * ROWS_PER_TILE, LAST_TILE_ROWS)],
                        agg_sh.at[pl.ds((NS - 1) * ROWS_PER_TILE, LAST_TILE_ROWS)])

    @pl.when(sid == 0)
    def _zero_deg():
        pltpu.sync_copy(zdeg_hbm, deg_sh)

    @pl.when(sid == 1)
    def _stage_rel():
        pltpu.sync_copy(rel_hbm, rel_sp.at[pl.ds(0, N_RELS)])

    for k in range(CH // LANES):
        ones_v[pl.ds(k * LANES, LANES)] = jnp.full((LANES,), 1.0, jnp.float32)

    plsc.subcore_barrier()

    def super_body(s, carry):
        c0 = s * SUP
        pltpu.sync_copy(src_hbm.at[wid, pl.ds(c0, SUP)], idx_src)
        pltpu.sync_copy(dst_hbm.at[wid, pl.ds(c0, SUP)], idx_dst)
        pltpu.sync_copy(typ_hbm.at[wid, pl.ds(c0, SUP)], idx_typ)
        n_inner = jnp.minimum(SUP, N_CHUNKS - c0)

        def mul_inplace(dst_rows, rel_rows):
            def mul_body(r, c2):
                for k in range(D // LANES):
                    sl = pl.ds(k * LANES, LANES)
                    dst_rows[r, sl] = dst_rows[r, sl] * rel_rows[r, sl]
                return c2
            lax.fori_loop(0, CH, mul_body, 0)

        def pair_body(j, c1):
            i0 = 2 * j
            i1 = i0 + 1
            ga = pltpu.async_copy(emb_hbm.at[idx_src.at[i0]], emb_a, sem_ga)
            ra = pltpu.async_copy(rel_sp.at[idx_typ.at[i0]], rel_a, sem_ra)
            gb = pltpu.async_copy(emb_hbm.at[idx_src.at[i1]], emb_b, sem_gb)
            rb = pltpu.async_copy(rel_sp.at[idx_typ.at[i1]], rel_b, sem_rb)
            ga.wait()
            ra.wait()
            mul_inplace(emb_a, rel_a)
            sa = pltpu.async_copy(emb_a, agg_sh.at[idx_dst.at[i0]],
                                  sem_s, add=True)
            da = pltpu.async_copy(ones_v, deg_sh.at[idx_dst.at[i0]],
                                  sem_d, add=True)
            gb.wait()
            rb.wait()
            mul_inplace(emb_b, rel_b)
            sa.wait()
            da.wait()
            sb = pltpu.async_copy(emb_b, agg_sh.at[idx_dst.at[i1]],
                                  sem_s, add=True)
            db = pltpu.async_copy(ones_v, deg_sh.at[idx_dst.at[i1]],
                                  sem_d, add=True)
            sb.wait()
            db.wait()
            return c1

        lax.fori_loop(0, n_inner // 2, pair_body, 0)

        @pl.when(n_inner % 2 == 1)
        def _tail():
            i0 = n_inner - 1
            ga = pltpu.async_copy(emb_hbm.at[idx_src.at[i0]], emb_a, sem_ga)
            ra = pltpu.async_copy(rel_sp.at[idx_typ.at[i0]], rel_a, sem_ra)
            ga.wait()
            ra.wait()
            mul_inplace(emb_a, rel_a)
            pltpu.sync_copy(emb_a, agg_sh.at[idx_dst.at[i0]], add=True)
            pltpu.sync_copy(ones_v, deg_sh.at[idx_dst.at[i0]], add=True)

        return carry

    lax.fori_loop(0, N_SUP, super_body, 0)

    plsc.subcore_barrier()

    @pl.when(sid < NS - 1)
    def _write_agg():
        pltpu.sync_copy(agg_sh.at[pl.ds(row0, ROWS_PER_TILE)],
                        agg_out.at[cid, pl.ds(row0, ROWS_PER_TILE)])

    @pl.when(sid == NS - 1)
    def _write_agg_last():
        pltpu.sync_copy(agg_sh.at[pl.ds((NS - 1) * ROWS_PER_TILE, LAST_TILE_ROWS)],
                        agg_out.at[cid, pl.ds((NS - 1) * ROWS_PER_TILE, LAST_TILE_ROWS)])

    @pl.when(sid == 0)
    def _write_deg():
        pltpu.sync_copy(deg_sh, deg_out.at[cid])


_sc_agg = functools.partial(
    pl.kernel,
    out_type=[
        jax.ShapeDtypeStruct((NC, N_NODES, D), jnp.float32),
        jax.ShapeDtypeStruct((NC, N_NODES), jnp.float32),
    ],
    mesh=plsc.VectorSubcoreMesh(core_axis_name="c", subcore_axis_name="s"),
    scratch_types=[
        pltpu.VMEM((SUP, CH), jnp.int32),
        pltpu.VMEM((SUP, CH), jnp.int32),
        pltpu.VMEM((SUP, CH), jnp.int32),
        pltpu.VMEM((CH, D), jnp.float32),
        pltpu.VMEM((CH, D), jnp.float32),
        pltpu.VMEM((CH, D), jnp.float32),
        pltpu.VMEM((CH, D), jnp.float32),
        pltpu.VMEM((CH,), jnp.float32),
        pltpu.VMEM_SHARED((N_NODES, D), jnp.float32),
        pltpu.VMEM_SHARED((N_NODES,), jnp.float32),
        pltpu.VMEM_SHARED((N_RELS + 4, D), jnp.float32),
        pltpu.SemaphoreType.DMA,
        pltpu.SemaphoreType.DMA,
        pltpu.SemaphoreType.DMA,
        pltpu.SemaphoreType.DMA,
        pltpu.SemaphoreType.DMA,
        pltpu.SemaphoreType.DMA,
    ],
)(_sc_agg_body)


ROW_BLK = 1000


def _tc_finish_body(aggp_ref, degp_ref, emb_ref, w_ref, wl_ref, b_ref, out_ref):
    agg = aggp_ref[0] + aggp_ref[1]
    deg = degp_ref[0, :, 0] + degp_ref[1, :, 0]
    norm = 1.0 / jnp.maximum(deg, 1.0)
    x = agg * norm[:, None]
    acc = jnp.dot(x, w_ref[...], preferred_element_type=jnp.float32)
    acc = acc + jnp.dot(emb_ref[...], wl_ref[...],
                        preferred_element_type=jnp.float32)
    out_ref[...] = jnp.tanh(acc + b_ref[...])


def _tc_finish(aggp, degp, emb, W, W_loop, b2d):
    grid = (N_NODES // ROW_BLK,)
    return pl.pallas_call(
        _tc_finish_body,
        grid=grid,
        in_specs=[
            pl.BlockSpec((NC, ROW_BLK, D), lambda i: (0, i, 0)),
            pl.BlockSpec((NC, ROW_BLK, 1), lambda i: (0, i, 0)),
            pl.BlockSpec((ROW_BLK, D), lambda i: (i, 0)),
            pl.BlockSpec((D, D), lambda i: (0, 0)),
            pl.BlockSpec((D, D), lambda i: (0, 0)),
            pl.BlockSpec((1, D), lambda i: (0, 0)),
        ],
        out_specs=pl.BlockSpec((ROW_BLK, D), lambda i: (i, 0)),
        out_shape=jax.ShapeDtypeStruct((N_NODES, D), jnp.float32),
    )(aggp, degp.reshape(NC, N_NODES, 1), emb, W, W_loop, b2d)


def _slab(x):
    x = x.reshape(NW, E_PER_TILE)
    x = jnp.pad(x, ((0, 0), (0, SLAB_CHUNKS * CH - E_PER_TILE)))
    return x.reshape(NW, SLAB_CHUNKS, CH)


def kernel(t, emb, edge_index, edge_type, W, W_loop, rel_emb, b):
    src = _slab(edge_index[0])
    dst = _slab(edge_index[1])
    typ = _slab(edge_type)
    zagg = jnp.zeros((N_NODES, D), jnp.float32)
    zdeg = jnp.zeros((N_NODES,), jnp.float32)
    aggp, degp = _sc_agg(src, dst, typ, emb, rel_emb, zagg, zdeg)
    return _tc_finish(aggp, degp, emb, W, W_loop, b.reshape(1, D))


# speedup vs baseline: 2.0533x; 1.0967x over previous
import functools

import jax
import jax.numpy as jnp
from jax import lax
from jax.experimental import pallas as pl
from jax.experimental.pallas import tpu as pltpu
from jax.experimental.pallas import tpu_sc as plsc

N_NODES = 10000
N_EDGES = 320000
N_RELS = 100
D = 128
LANES = 16

NC = 2
NS = 16
NW = NC * NS
CH = 80
N_CHUNKS = 125
SLAB_CHUNKS = 128
SUP = 16
N_SUP = 8
E_PER_TILE = N_CHUNKS * CH
ROWS_PER_TILE = 624
LAST_TILE_ROWS = N_NODES - (NS - 1) * ROWS_PER_TILE


def _sc_agg_body(src_hbm, dst_hbm, typ_hbm, emb_hbm, rel_hbm, zagg_hbm, zdeg_hbm,
                 agg_out, deg_out,
                 idx_src, idx_dst, idx_typ, emb_a, emb_b, rel_a, rel_b, ones_v,
                 agg_sh, deg_sh, rel_sp,
                 sem_ga, sem_gb, sem_ra, sem_rb, sem_s, sem_d):
    cid = lax.axis_index("c")
    sid = lax.axis_index("s")
    wid = cid * NS + sid

    row0 = sid * ROWS_PER_TILE

    @pl.when(sid < NS - 1)
    def _zero_agg():
        pltpu.sync_copy(zagg_hbm.at[pl.ds(row0, ROWS_PER_TILE)],
                        agg_sh.at[pl.ds(row0, ROWS_PER_TILE)])

    @pl.when(sid == NS - 1)
    def _zero_agg_last():
        pltpu.sync_copy(zagg_hbm.at[pl.ds((NS - 1) * ROWS_PER_TILE, LAST_TILE_ROWS)],
                        agg_sh.at[pl.ds((NS - 1) * ROWS_PER_TILE, LAST_TILE_ROWS)])

    @pl.when(sid == 0)
    def _zero_deg():
        pltpu.sync_copy(zdeg_hbm, deg_sh)

    @pl.when(sid == 1)
    def _stage_rel():
        pltpu.sync_copy(rel_hbm, rel_sp.at[pl.ds(0, N_RELS)])

    for k in range(CH // LANES):
        ones_v[pl.ds(k * LANES, LANES)] = jnp.full((LANES,), 1.0, jnp.float32)

    plsc.subcore_barrier()

    def super_body(s, carry):
        c0 = s * SUP
        pltpu.sync_copy(src_hbm.at[wid, pl.ds(c0, SUP)], idx_src)
        pltpu.sync_copy(dst_hbm.at[wid, pl.ds(c0, SUP)], idx_dst)
        pltpu.sync_copy(typ_hbm.at[wid, pl.ds(c0, SUP)], idx_typ)
        n_inner = jnp.minimum(SUP, N_CHUNKS - c0)

        def mul_inplace(dst_rows, rel_rows):
            def mul_body(r, c2):
                for k in range(D // LANES):
                    sl = pl.ds(k * LANES, LANES)
                    dst_rows[r, sl] = dst_rows[r, sl] * rel_rows[r, sl]
                return c2
            lax.fori_loop(0, CH, mul_body, 0)

        def pair_body(j, c1):
            i0 = 2 * j
            i1 = i0 + 1
            ga = pltpu.async_copy(emb_hbm.at[idx_src.at[i0]], emb_a, sem_ga)
            ra = pltpu.async_copy(rel_sp.at[idx_typ.at[i0]], rel_a, sem_ra)

            @pl.when(j > 0)
            def _drain_prev():
                pltpu.make_async_copy(emb_b, agg_sh.at[pl.ds(0, CH)],
                                      sem_s).wait()
                pltpu.make_async_copy(ones_v, deg_sh.at[pl.ds(0, CH)],
                                      sem_d).wait()

            gb = pltpu.async_copy(emb_hbm.at[idx_src.at[i1]], emb_b, sem_gb)
            rb = pltpu.async_copy(rel_sp.at[idx_typ.at[i1]], rel_b, sem_rb)
            ga.wait()
            ra.wait()
            mul_inplace(emb_a, rel_a)
            sa = pltpu.async_copy(emb_a, agg_sh.at[idx_dst.at[i0]],
                                  sem_s, add=True)
            da = pltpu.async_copy(ones_v, deg_sh.at[idx_dst.at[i0]],
                                  sem_d, add=True)
            gb.wait()
            rb.wait()
            mul_inplace(emb_b, rel_b)
            sa.wait()
            da.wait()
            pltpu.async_copy(emb_b, agg_sh.at[idx_dst.at[i1]],
                             sem_s, add=True)
            pltpu.async_copy(ones_v, deg_sh.at[idx_dst.at[i1]],
                             sem_d, add=True)
            return c1

        lax.fori_loop(0, n_inner // 2, pair_body, 0)

        pltpu.make_async_copy(emb_b, agg_sh.at[pl.ds(0, CH)], sem_s).wait()
        pltpu.make_async_copy(ones_v, deg_sh.at[pl.ds(0, CH)], sem_d).wait()

        @pl.when(n_inner % 2 == 1)
        def _tail():
            i0 = n_inner - 1
            ga = pltpu.async_copy(emb_hbm.at[idx_src.at[i0]], emb_a, sem_ga)
            ra = pltpu.async_copy(rel_sp.at[idx_typ.at[i0]], rel_a, sem_ra)
            ga.wait()
            ra.wait()
            mul_inplace(emb_a, rel_a)
            pltpu.sync_copy(emb_a, agg_sh.at[idx_dst.at[i0]], add=True)
            pltpu.sync_copy(ones_v, deg_sh.at[idx_dst.at[i0]], add=True)

        return carry

    lax.fori_loop(0, N_SUP, super_body, 0)

    plsc.subcore_barrier()

    @pl.when(sid < NS - 1)
    def _write_agg():
        pltpu.sync_copy(agg_sh.at[pl.ds(row0, ROWS_PER_TILE)],
                        agg_out.at[cid, pl.ds(row0, ROWS_PER_TILE)])

    @pl.when(sid == NS - 1)
    def _write_agg_last():
        pltpu.sync_copy(agg_sh.at[pl.ds((NS - 1) * ROWS_PER_TILE, LAST_TILE_ROWS)],
                        agg_out.at[cid, pl.ds((NS - 1) * ROWS_PER_TILE, LAST_TILE_ROWS)])

    @pl.when(sid == 0)
    def _write_deg():
        pltpu.sync_copy(deg_sh, deg_out.at[cid])


_sc_agg = functools.partial(
    pl.kernel,
    out_type=[
        jax.ShapeDtypeStruct((NC, N_NODES, D), jnp.float32),
        jax.ShapeDtypeStruct((NC, N_NODES), jnp.float32),
    ],
    mesh=plsc.VectorSubcoreMesh(core_axis_name="c", subcore_axis_name="s"),
    scratch_types=[
        pltpu.VMEM((SUP, CH), jnp.int32),
        pltpu.VMEM((SUP, CH), jnp.int32),
        pltpu.VMEM((SUP, CH), jnp.int32),
        pltpu.VMEM((CH, D), jnp.float32),
        pltpu.VMEM((CH, D), jnp.float32),
        pltpu.VMEM((CH, D), jnp.float32),
        pltpu.VMEM((CH, D), jnp.float32),
        pltpu.VMEM((CH,), jnp.float32),
        pltpu.VMEM_SHARED((N_NODES, D), jnp.float32),
        pltpu.VMEM_SHARED((N_NODES,), jnp.float32),
        pltpu.VMEM_SHARED((N_RELS + 4, D), jnp.float32),
        pltpu.SemaphoreType.DMA,
        pltpu.SemaphoreType.DMA,
        pltpu.SemaphoreType.DMA,
        pltpu.SemaphoreType.DMA,
        pltpu.SemaphoreType.DMA,
        pltpu.SemaphoreType.DMA,
    ],
)(_sc_agg_body)


ROW_BLK = 1000


def _tc_finish_body(aggp_ref, degp_ref, emb_ref, w_ref, wl_ref, b_ref, out_ref):
    agg = aggp_ref[0] + aggp_ref[1]
    deg = degp_ref[0, :, 0] + degp_ref[1, :, 0]
    norm = 1.0 / jnp.maximum(deg, 1.0)
    x = agg * norm[:, None]
    acc = jnp.dot(x, w_ref[...], preferred_element_type=jnp.float32)
    acc = acc + jnp.dot(emb_ref[...], wl_ref[...],
                        preferred_element_type=jnp.float32)
    out_ref[...] = jnp.tanh(acc + b_ref[...])


def _tc_finish(aggp, degp, emb, W, W_loop, b2d):
    grid = (N_NODES // ROW_BLK,)
    return pl.pallas_call(
        _tc_finish_body,
        grid=grid,
        in_specs=[
            pl.BlockSpec((NC, ROW_BLK, D), lambda i: (0, i, 0)),
            pl.BlockSpec((NC, ROW_BLK, 1), lambda i: (0, i, 0)),
            pl.BlockSpec((ROW_BLK, D), lambda i: (i, 0)),
            pl.BlockSpec((D, D), lambda i: (0, 0)),
            pl.BlockSpec((D, D), lambda i: (0, 0)),
            pl.BlockSpec((1, D), lambda i: (0, 0)),
        ],
        out_specs=pl.BlockSpec((ROW_BLK, D), lambda i: (i, 0)),
        out_shape=jax.ShapeDtypeStruct((N_NODES, D), jnp.float32),
    )(aggp, degp.reshape(NC, N_NODES, 1), emb, W, W_loop, b2d)


def _slab(x):
    x = x.reshape(NW, E_PER_TILE)
    x = jnp.pad(x, ((0, 0), (0, SLAB_CHUNKS * CH - E_PER_TILE)))
    return x.reshape(NW, SLAB_CHUNKS, CH)


def kernel(t, emb, edge_index, edge_type, W, W_loop, rel_emb, b):
    src = _slab(edge_index[0])
    dst = _slab(edge_index[1])
    typ = _slab(edge_type)
    zagg = jnp.zeros((N_NODES, D), jnp.float32)
    zdeg = jnp.zeros((N_NODES,), jnp.float32)
    aggp, degp = _sc_agg(src, dst, typ, emb, rel_emb, zagg, zdeg)
    return _tc_finish(aggp, degp, emb, W, W_loop, b.reshape(1, D))


# speedup vs baseline: 2.0595x; 1.0031x over previous
import functools

import jax
import jax.numpy as jnp
from jax import lax
from jax.experimental import pallas as pl
from jax.experimental.pallas import tpu as pltpu
from jax.experimental.pallas import tpu_sc as plsc

N_NODES = 10000
N_EDGES = 320000
N_RELS = 100
D = 128
LANES = 16

NC = 2
NS = 16
NW = NC * NS
CH = 80
N_CHUNKS = 125
SLAB_CHUNKS = 128
SUP = 8
N_SUP = 16
E_PER_TILE = N_CHUNKS * CH
ROWS_PER_TILE = 624
LAST_TILE_ROWS = N_NODES - (NS - 1) * ROWS_PER_TILE


def _sc_agg_body(src_hbm, dst_hbm, typ_hbm, emb_hbm, rel_hbm, zagg_hbm, zdeg_hbm,
                 agg_out, deg_out,
                 idx_src, idx_dst, idx_typ, emb_a, emb_b, rel_a, rel_b, ones_v,
                 agg_sh, deg_sh, rel_sp,
                 sem_ga, sem_gb, sem_ra, sem_rb, sem_s, sem_d, sem_i):
    cid = lax.axis_index("c")
    sid = lax.axis_index("s")
    wid = cid * NS + sid

    row0 = sid * ROWS_PER_TILE

    @pl.when(sid < NS - 1)
    def _zero_agg():
        pltpu.sync_copy(zagg_hbm.at[pl.ds(row0, ROWS_PER_TILE)],
                        agg_sh.at[pl.ds(row0, ROWS_PER_TILE)])

    @pl.when(sid == NS - 1)
    def _zero_agg_last():
        pltpu.sync_copy(zagg_hbm.at[pl.ds((NS - 1) * ROWS_PER_TILE, LAST_TILE_ROWS)],
                        agg_sh.at[pl.ds((NS - 1) * ROWS_PER_TILE, LAST_TILE_ROWS)])

    @pl.when(sid == 0)
    def _zero_deg():
        pltpu.sync_copy(zdeg_hbm, deg_sh)

    @pl.when(sid == 1)
    def _stage_rel():
        pltpu.sync_copy(rel_hbm, rel_sp.at[pl.ds(0, N_RELS)])

    for k in range(CH // LANES):
        ones_v[pl.ds(k * LANES, LANES)] = jnp.full((LANES,), 1.0, jnp.float32)

    plsc.subcore_barrier()

    pltpu.sync_copy(src_hbm.at[wid, pl.ds(0, SUP)], idx_src.at[0])
    pltpu.sync_copy(dst_hbm.at[wid, pl.ds(0, SUP)], idx_dst.at[0])
    pltpu.sync_copy(typ_hbm.at[wid, pl.ds(0, SUP)], idx_typ.at[0])

    def super_body(s, carry):
        c0 = s * SUP
        slot = lax.rem(s, 2)
        nslot = 1 - slot
        n_inner = jnp.minimum(SUP, N_CHUNKS - c0)
        idx_src_s = idx_src.at[slot]
        idx_dst_s = idx_dst.at[slot]
        idx_typ_s = idx_typ.at[slot]

        @pl.when(s + 1 < N_SUP)
        def _prefetch_idx():
            c1 = c0 + SUP
            pltpu.async_copy(src_hbm.at[wid, pl.ds(c1, SUP)],
                             idx_src.at[nslot], sem_i)
            pltpu.async_copy(dst_hbm.at[wid, pl.ds(c1, SUP)],
                             idx_dst.at[nslot], sem_i)
            pltpu.async_copy(typ_hbm.at[wid, pl.ds(c1, SUP)],
                             idx_typ.at[nslot], sem_i)

        def mul_inplace(dst_rows, rel_rows):
            def mul_body(r, c2):
                for k in range(D // LANES):
                    sl = pl.ds(k * LANES, LANES)
                    dst_rows[r, sl] = dst_rows[r, sl] * rel_rows[r, sl]
                return c2
            lax.fori_loop(0, CH, mul_body, 0)

        def pair_body(j, c1):
            i0 = 2 * j
            i1 = i0 + 1
            ga = pltpu.async_copy(emb_hbm.at[idx_src_s.at[i0]], emb_a, sem_ga)
            ra = pltpu.async_copy(rel_sp.at[idx_typ_s.at[i0]], rel_a, sem_ra)

            @pl.when(j > 0)
            def _drain_prev():
                pltpu.make_async_copy(emb_b, agg_sh.at[pl.ds(0, CH)],
                                      sem_s).wait()
                pltpu.make_async_copy(ones_v, deg_sh.at[pl.ds(0, CH)],
                                      sem_d).wait()

            gb = pltpu.async_copy(emb_hbm.at[idx_src_s.at[i1]], emb_b, sem_gb)
            rb = pltpu.async_copy(rel_sp.at[idx_typ_s.at[i1]], rel_b, sem_rb)
            ga.wait()
            ra.wait()
            mul_inplace(emb_a, rel_a)
            sa = pltpu.async_copy(emb_a, agg_sh.at[idx_dst_s.at[i0]],
                                  sem_s, add=True)
            da = pltpu.async_copy(ones_v, deg_sh.at[idx_dst_s.at[i0]],
                                  sem_d, add=True)
            gb.wait()
            rb.wait()
            mul_inplace(emb_b, rel_b)
            sa.wait()
            da.wait()
            pltpu.async_copy(emb_b, agg_sh.at[idx_dst_s.at[i1]],
                             sem_s, add=True)
            pltpu.async_copy(ones_v, deg_sh.at[idx_dst_s.at[i1]],
                             sem_d, add=True)
            return c1

        lax.fori_loop(0, n_inner // 2, pair_body, 0)

        pltpu.make_async_copy(emb_b, agg_sh.at[pl.ds(0, CH)], sem_s).wait()
        pltpu.make_async_copy(ones_v, deg_sh.at[pl.ds(0, CH)], sem_d).wait()

        @pl.when(n_inner % 2 == 1)
        def _tail():
            i0 = n_inner - 1
            ga = pltpu.async_copy(emb_hbm.at[idx_src_s.at[i0]], emb_a, sem_ga)
            ra = pltpu.async_copy(rel_sp.at[idx_typ_s.at[i0]], rel_a, sem_ra)
            ga.wait()
            ra.wait()
            mul_inplace(emb_a, rel_a)
            pltpu.sync_copy(emb_a, agg_sh.at[idx_dst_s.at[i0]], add=True)
            pltpu.sync_copy(ones_v, deg_sh.at[idx_dst_s.at[i0]], add=True)

        @pl.when(s + 1 < N_SUP)
        def _wait_idx():
            for ref in (idx_src, idx_dst, idx_typ):
                pltpu.make_async_copy(src_hbm.at[wid, pl.ds(0, SUP)],
                                      ref.at[nslot], sem_i).wait()

        return carry

    lax.fori_loop(0, N_SUP, super_body, 0)

    plsc.subcore_barrier()

    @pl.when(sid < NS - 1)
    def _write_agg():
        pltpu.sync_copy(agg_sh.at[pl.ds(row0, ROWS_PER_TILE)],
                        agg_out.at[cid, pl.ds(row0, ROWS_PER_TILE)])

    @pl.when(sid == NS - 1)
    def _write_agg_last():
        pltpu.sync_copy(agg_sh.at[pl.ds((NS - 1) * ROWS_PER_TILE, LAST_TILE_ROWS)],
                        agg_out.at[cid, pl.ds((NS - 1) * ROWS_PER_TILE, LAST_TILE_ROWS)])

    @pl.when(sid == 0)
    def _write_deg():
        pltpu.sync_copy(deg_sh, deg_out.at[cid])


_sc_agg = functools.partial(
    pl.kernel,
    out_type=[
        jax.ShapeDtypeStruct((NC, N_NODES, D), jnp.float32),
        jax.ShapeDtypeStruct((NC, N_NODES), jnp.float32),
    ],
    mesh=plsc.VectorSubcoreMesh(core_axis_name="c", subcore_axis_name="s"),
    scratch_types=[
        pltpu.VMEM((2, SUP, CH), jnp.int32),
        pltpu.VMEM((2, SUP, CH), jnp.int32),
        pltpu.VMEM((2, SUP, CH), jnp.int32),
        pltpu.VMEM((CH, D), jnp.float32),
        pltpu.VMEM((CH, D), jnp.float32),
        pltpu.VMEM((CH, D), jnp.float32),
        pltpu.VMEM((CH, D), jnp.float32),
        pltpu.VMEM((CH,), jnp.float32),
        pltpu.VMEM_SHARED((N_NODES, D), jnp.float32),
        pltpu.VMEM_SHARED((N_NODES,), jnp.float32),
        pltpu.VMEM_SHARED((N_RELS + 4, D), jnp.float32),
        pltpu.SemaphoreType.DMA,
        pltpu.SemaphoreType.DMA,
        pltpu.SemaphoreType.DMA,
        pltpu.SemaphoreType.DMA,
        pltpu.SemaphoreType.DMA,
        pltpu.SemaphoreType.DMA,
        pltpu.SemaphoreType.DMA,
    ],
)(_sc_agg_body)


ROW_BLK = 1000


def _tc_finish_body(aggp_ref, degp_ref, emb_ref, w_ref, wl_ref, b_ref, out_ref):
    agg = aggp_ref[0] + aggp_ref[1]
    deg = degp_ref[0, :, 0] + degp_ref[1, :, 0]
    norm = 1.0 / jnp.maximum(deg, 1.0)
    x = agg * norm[:, None]
    acc = jnp.dot(x, w_ref[...], preferred_element_type=jnp.float32)
    acc = acc + jnp.dot(emb_ref[...], wl_ref[...],
                        preferred_element_type=jnp.float32)
    out_ref[...] = jnp.tanh(acc + b_ref[...])


def _tc_finish(aggp, degp, emb, W, W_loop, b2d):
    grid = (N_NODES // ROW_BLK,)
    return pl.pallas_call(
        _tc_finish_body,
        grid=grid,
        in_specs=[
            pl.BlockSpec((NC, ROW_BLK, D), lambda i: (0, i, 0)),
            pl.BlockSpec((NC, ROW_BLK, 1), lambda i: (0, i, 0)),
            pl.BlockSpec((ROW_BLK, D), lambda i: (i, 0)),
            pl.BlockSpec((D, D), lambda i: (0, 0)),
            pl.BlockSpec((D, D), lambda i: (0, 0)),
            pl.BlockSpec((1, D), lambda i: (0, 0)),
        ],
        out_specs=pl.BlockSpec((ROW_BLK, D), lambda i: (i, 0)),
        out_shape=jax.ShapeDtypeStruct((N_NODES, D), jnp.float32),
    )(aggp, degp.reshape(NC, N_NODES, 1), emb, W, W_loop, b2d)


def _slab(x):
    x = x.reshape(NW, E_PER_TILE)
    x = jnp.pad(x, ((0, 0), (0, SLAB_CHUNKS * CH - E_PER_TILE)))
    return x.reshape(NW, SLAB_CHUNKS, CH)


def kernel(t, emb, edge_index, edge_type, W, W_loop, rel_emb, b):
    src = _slab(edge_index[0])
    dst = _slab(edge_index[1])
    typ = _slab(edge_type)
    zagg = jnp.zeros((N_NODES, D), jnp.float32)
    zdeg = jnp.zeros((N_NODES,), jnp.float32)
    aggp, degp = _sc_agg(src, dst, typ, emb, rel_emb, zagg, zdeg)
    return _tc_finish(aggp, degp, emb, W, W_loop, b.reshape(1, D))


# speedup vs baseline: 2.2144x; 1.0752x over previous
import functools

import jax
import jax.numpy as jnp
from jax import lax
from jax.experimental import pallas as pl
from jax.experimental.pallas import tpu as pltpu
from jax.experimental.pallas import tpu_sc as plsc

N_NODES = 10000
N_EDGES = 320000
N_RELS = 100
D = 128
LANES = 16

NC = 2
NS = 16
NW = NC * NS
CH = 80
N_CHUNKS = 125
SLAB_CHUNKS = 128
SUP = 8
N_SUP = 16
E_PER_TILE = N_CHUNKS * CH
ROWS_PER_TILE = 624
LAST_TILE_ROWS = N_NODES - (NS - 1) * ROWS_PER_TILE


def _sc_agg_body(src_hbm, dst_hbm, typ_hbm, emb_hbm, rel_hbm, zagg_hbm, zdeg_hbm,
                 agg_out, deg_out,
                 idx_src, idx_dst, idx_typ, emb_a, emb_b, rel_a, rel_b, ones_v,
                 agg_sh, deg_sh, rel_sp,
                 sem_ga, sem_gb, sem_ra, sem_rb, sem_s, sem_d, sem_i):
    cid = lax.axis_index("c")
    sid = lax.axis_index("s")
    wid = cid * NS + sid

    row0 = sid * ROWS_PER_TILE

    @pl.when(sid < NS - 1)
    def _zero_agg():
        pltpu.sync_copy(zagg_hbm.at[pl.ds(row0, ROWS_PER_TILE)],
                        agg_sh.at[pl.ds(row0, ROWS_PER_TILE)])

    @pl.when(sid == NS - 1)
    def _zero_agg_last():
        pltpu.sync_copy(zagg_hbm.at[pl.ds((NS - 1) * ROWS_PER_TILE, LAST_TILE_ROWS)],
                        agg_sh.at[pl.ds((NS - 1) * ROWS_PER_TILE, LAST_TILE_ROWS)])

    @pl.when(sid == 0)
    def _zero_deg():
        pltpu.sync_copy(zdeg_hbm, deg_sh)

    @pl.when(sid == 1)
    def _stage_rel():
        pltpu.sync_copy(rel_hbm, rel_sp.at[pl.ds(0, N_RELS)])

    for k in range(CH // LANES):
        ones_v[pl.ds(k * LANES, LANES)] = jnp.full((LANES,), 1.0, jnp.float32)

    plsc.subcore_barrier()

    pltpu.sync_copy(src_hbm.at[wid, pl.ds(0, SUP)], idx_src.at[0])
    pltpu.sync_copy(dst_hbm.at[wid, pl.ds(0, SUP)], idx_dst.at[0])
    pltpu.sync_copy(typ_hbm.at[wid, pl.ds(0, SUP)], idx_typ.at[0])

    def mul_inplace(dst_rows, rel_rows):
        def mul_body(r, c2):
            for k in range(D // LANES):
                sl = pl.ds(k * LANES, LANES)
                dst_rows[r, sl] = dst_rows[r, sl] * rel_rows[r, sl]
            return c2
        lax.fori_loop(0, CH, mul_body, 0)

    PAIRS_PER_SUP = SUP // 2
    N_PAIRS = (N_CHUNKS - 1) // 2

    def pair_body(j, carry):
        sidx = j // PAIRS_PER_SUP
        slot = lax.rem(sidx, 2)
        nslot = 1 - slot
        jj = lax.rem(j, PAIRS_PER_SUP)
        i0 = 2 * jj
        i1 = i0 + 1
        idx_src_s = idx_src.at[slot]
        idx_dst_s = idx_dst.at[slot]
        idx_typ_s = idx_typ.at[slot]
        first_of_sup = jj == 0

        @pl.when(jnp.logical_and(first_of_sup, sidx > 0))
        def _wait_idx():
            for ref in (idx_src, idx_dst, idx_typ):
                pltpu.make_async_copy(src_hbm.at[wid, pl.ds(0, SUP)],
                                      ref.at[slot], sem_i).wait()

        @pl.when(jnp.logical_and(first_of_sup, sidx + 1 < N_SUP))
        def _prefetch_idx():
            c1 = (sidx + 1) * SUP
            pltpu.async_copy(src_hbm.at[wid, pl.ds(c1, SUP)],
                             idx_src.at[nslot], sem_i)
            pltpu.async_copy(dst_hbm.at[wid, pl.ds(c1, SUP)],
                             idx_dst.at[nslot], sem_i)
            pltpu.async_copy(typ_hbm.at[wid, pl.ds(c1, SUP)],
                             idx_typ.at[nslot], sem_i)

        ga = pltpu.async_copy(emb_hbm.at[idx_src_s.at[i0]], emb_a, sem_ga)
        ra = pltpu.async_copy(rel_sp.at[idx_typ_s.at[i0]], rel_a, sem_ra)

        @pl.when(j > 0)
        def _drain_prev():
            pltpu.make_async_copy(emb_b, agg_sh.at[pl.ds(0, CH)],
                                  sem_s).wait()
            pltpu.make_async_copy(ones_v, deg_sh.at[pl.ds(0, CH)],
                                  sem_d).wait()

        gb = pltpu.async_copy(emb_hbm.at[idx_src_s.at[i1]], emb_b, sem_gb)
        rb = pltpu.async_copy(rel_sp.at[idx_typ_s.at[i1]], rel_b, sem_rb)
        ga.wait()
        ra.wait()
        mul_inplace(emb_a, rel_a)
        sa = pltpu.async_copy(emb_a, agg_sh.at[idx_dst_s.at[i0]],
                              sem_s, add=True)
        da = pltpu.async_copy(ones_v, deg_sh.at[idx_dst_s.at[i0]],
                              sem_d, add=True)
        gb.wait()
        rb.wait()
        mul_inplace(emb_b, rel_b)
        sa.wait()
        da.wait()
        pltpu.async_copy(emb_b, agg_sh.at[idx_dst_s.at[i1]],
                         sem_s, add=True)
        pltpu.async_copy(ones_v, deg_sh.at[idx_dst_s.at[i1]],
                         sem_d, add=True)
        return carry

    lax.fori_loop(0, N_PAIRS, pair_body, 0)

    pltpu.make_async_copy(emb_b, agg_sh.at[pl.ds(0, CH)], sem_s).wait()
    pltpu.make_async_copy(ones_v, deg_sh.at[pl.ds(0, CH)], sem_d).wait()

    t_src = idx_src.at[(N_SUP - 1) % 2]
    t_dst = idx_dst.at[(N_SUP - 1) % 2]
    t_typ = idx_typ.at[(N_SUP - 1) % 2]
    t_row = N_CHUNKS - 1 - (N_SUP - 1) * SUP
    ga = pltpu.async_copy(emb_hbm.at[t_src.at[t_row]], emb_a, sem_ga)
    ra = pltpu.async_copy(rel_sp.at[t_typ.at[t_row]], rel_a, sem_ra)
    ga.wait()
    ra.wait()
    mul_inplace(emb_a, rel_a)
    pltpu.sync_copy(emb_a, agg_sh.at[t_dst.at[t_row]], add=True)
    pltpu.sync_copy(ones_v, deg_sh.at[t_dst.at[t_row]], add=True)

    plsc.subcore_barrier()

    @pl.when(sid < NS - 1)
    def _write_agg():
        pltpu.sync_copy(agg_sh.at[pl.ds(row0, ROWS_PER_TILE)],
                        agg_out.at[cid, pl.ds(row0, ROWS_PER_TILE)])

    @pl.when(sid == NS - 1)
    def _write_agg_last():
        pltpu.sync_copy(agg_sh.at[pl.ds((NS - 1) * ROWS_PER_TILE, LAST_TILE_ROWS)],
                        agg_out.at[cid, pl.ds((NS - 1) * ROWS_PER_TILE, LAST_TILE_ROWS)])

    @pl.when(sid == 0)
    def _write_deg():
        pltpu.sync_copy(deg_sh, deg_out.at[cid])


_sc_agg = functools.partial(
    pl.kernel,
    out_type=[
        jax.ShapeDtypeStruct((NC, N_NODES, D), jnp.float32),
        jax.ShapeDtypeStruct((NC, N_NODES), jnp.float32),
    ],
    mesh=plsc.VectorSubcoreMesh(core_axis_name="c", subcore_axis_name="s"),
    scratch_types=[
        pltpu.VMEM((2, SUP, CH), jnp.int32),
        pltpu.VMEM((2, SUP, CH), jnp.int32),
        pltpu.VMEM((2, SUP, CH), jnp.int32),
        pltpu.VMEM((CH, D), jnp.float32),
        pltpu.VMEM((CH, D), jnp.float32),
        pltpu.VMEM((CH, D), jnp.float32),
        pltpu.VMEM((CH, D), jnp.float32),
        pltpu.VMEM((CH,), jnp.float32),
        pltpu.VMEM_SHARED((N_NODES, D), jnp.float32),
        pltpu.VMEM_SHARED((N_NODES,), jnp.float32),
        pltpu.VMEM_SHARED((N_RELS + 4, D), jnp.float32),
        pltpu.SemaphoreType.DMA,
        pltpu.SemaphoreType.DMA,
        pltpu.SemaphoreType.DMA,
        pltpu.SemaphoreType.DMA,
        pltpu.SemaphoreType.DMA,
        pltpu.SemaphoreType.DMA,
        pltpu.SemaphoreType.DMA,
    ],
)(_sc_agg_body)


ROW_BLK = 1000


def _tc_finish_body(aggp_ref, degp_ref, emb_ref, w_ref, wl_ref, b_ref, out_ref):
    agg = aggp_ref[0] + aggp_ref[1]
    deg = degp_ref[0, :, 0] + degp_ref[1, :, 0]
    norm = 1.0 / jnp.maximum(deg, 1.0)
    x = agg * norm[:, None]
    acc = jnp.dot(x, w_ref[...], preferred_element_type=jnp.float32)
    acc = acc + jnp.dot(emb_ref[...], wl_ref[...],
                        preferred_element_type=jnp.float32)
    out_ref[...] = jnp.tanh(acc + b_ref[...])


def _tc_finish(aggp, degp, emb, W, W_loop, b2d):
    grid = (N_NODES // ROW_BLK,)
    return pl.pallas_call(
        _tc_finish_body,
        grid=grid,
        in_specs=[
            pl.BlockSpec((NC, ROW_BLK, D), lambda i: (0, i, 0)),
            pl.BlockSpec((NC, ROW_BLK, 1), lambda i: (0, i, 0)),
            pl.BlockSpec((ROW_BLK, D), lambda i: (i, 0)),
            pl.BlockSpec((D, D), lambda i: (0, 0)),
            pl.BlockSpec((D, D), lambda i: (0, 0)),
            pl.BlockSpec((1, D), lambda i: (0, 0)),
        ],
        out_specs=pl.BlockSpec((ROW_BLK, D), lambda i: (i, 0)),
        out_shape=jax.ShapeDtypeStruct((N_NODES, D), jnp.float32),
    )(aggp, degp.reshape(NC, N_NODES, 1), emb, W, W_loop, b2d)


def _slab(x):
    x = x.reshape(NW, E_PER_TILE)
    x = jnp.pad(x, ((0, 0), (0, SLAB_CHUNKS * CH - E_PER_TILE)))
    return x.reshape(NW, SLAB_CHUNKS, CH)


def kernel(t, emb, edge_index, edge_type, W, W_loop, rel_emb, b):
    src = _slab(edge_index[0])
    dst = _slab(edge_index[1])
    typ = _slab(edge_type)
    zagg = jnp.zeros((N_NODES, D), jnp.float32)
    zdeg = jnp.zeros((N_NODES,), jnp.float32)
    aggp, degp = _sc_agg(src, dst, typ, emb, rel_emb, zagg, zdeg)
    return _tc_finish(aggp, degp, emb, W, W_loop, b.reshape(1, D))
